# baseline (speedup 1.0000x reference)
import jax
import jax.numpy as jnp
from jax import lax
from jax.experimental import pallas as pl
from jax.experimental.pallas import tpu as pltpu
from jax.experimental.pallas import tpu_sc as plsc

B, L, D = 4096, 200, 64
NC, NS = 2, 16
NW = NC * NS
BPW = B // NW
LANES = 16
DV = D // LANES


def _sc_gather_sum_body(item_hbm, user_hbm, idx_hbm, uids_hbm, out_hbm,
                        idx_v, uids_v, self_v, rows0_v, rows1_v, out_v,
                        sem0, sem1, usem):
    wid = lax.axis_index("s") * NC + lax.axis_index("c")
    base = wid * BPW

    pltpu.sync_copy(idx_hbm.at[pl.ds(base, BPW)], idx_v)
    pltpu.sync_copy(uids_hbm.at[pl.ds(base, BPW)], uids_v)
    user_cp = pltpu.async_copy(user_hbm.at[uids_v], self_v, usem)

    def start(b, rows_v, sem):
        pltpu.async_copy(item_hbm.at[idx_v.at[b]], rows_v, sem)

    def accum_row(b, rows_v):
        def accum(j, accs):
            return tuple(accs[d] + rows_v[j, pl.ds(d * LANES, LANES)]
                         for d in range(DV))

        init = tuple(self_v[b, pl.ds(d * LANES, LANES)] for d in range(DV))
        accs = lax.fori_loop(0, L, accum, init, unroll=8)
        for d in range(DV):
            out_v[b, pl.ds(d * LANES, LANES)] = accs[d]

    start(0, rows0_v, sem0)
    user_cp.wait()

    def outer(k, carry):
        start(2 * k + 1, rows1_v, sem1)
        pltpu.make_async_copy(
            item_hbm.at[idx_v.at[0]], rows0_v, sem0).wait()
        accum_row(2 * k, rows0_v)

        @pl.when(k < BPW // 2 - 1)
        def _():
            start(2 * k + 2, rows0_v, sem0)

        pltpu.make_async_copy(
            item_hbm.at[idx_v.at[0]], rows1_v, sem1).wait()
        accum_row(2 * k + 1, rows1_v)
        return carry

    lax.fori_loop(0, BPW // 2, outer, 0)
    pltpu.sync_copy(out_v, out_hbm.at[pl.ds(base, BPW)])


@jax.jit
def _sc_gather_sum(u_item_pad, uids, item_table, user_table):
    mesh = plsc.VectorSubcoreMesh(core_axis_name="c", subcore_axis_name="s")
    return pl.kernel(
        _sc_gather_sum_body,
        out_type=jax.ShapeDtypeStruct((B, D), jnp.float32),
        mesh=mesh,
        scratch_types=[
            pltpu.VMEM((BPW, L), jnp.int32),
            pltpu.VMEM((BPW,), jnp.int32),
            pltpu.VMEM((BPW, D), jnp.float32),
            pltpu.VMEM((L, D), jnp.float32),
            pltpu.VMEM((L, D), jnp.float32),
            pltpu.VMEM((BPW, D), jnp.float32),
            pltpu.SemaphoreType.DMA,
            pltpu.SemaphoreType.DMA,
            pltpu.SemaphoreType.DMA,
        ],
        compiler_params=pltpu.CompilerParams(use_tc_tiling_on_sc=False),
    )(item_table, user_table, u_item_pad, uids)


def _mm_body(s_ref, w_ref, b_ref, o_ref):
    x = s_ref[...]
    y = lax.dot_general(x, w_ref[...], (((1,), (1,)), ((), ())),
                        preferred_element_type=jnp.float32)
    y = y + b_ref[...]
    o_ref[...] = jnp.where(y > 0, y, jnp.exp(jnp.minimum(y, 0.0)) - 1.0)


@jax.jit
def _mm_elu(s, W, b2d):
    blk = 512
    return pl.pallas_call(
        _mm_body,
        grid=(B // blk,),
        in_specs=[
            pl.BlockSpec((blk, D), lambda i: (i, 0)),
            pl.BlockSpec((D, D), lambda i: (0, 0)),
            pl.BlockSpec((1, D), lambda i: (0, 0)),
        ],
        out_specs=pl.BlockSpec((blk, D), lambda i: (i, 0)),
        out_shape=jax.ShapeDtypeStruct((B, D), jnp.float32),
    )(s, W, b2d)


def kernel(uids, u_item_pad, item_table, user_table, W, b):
    s = _sc_gather_sum(u_item_pad.astype(jnp.int32), uids.astype(jnp.int32),
                       item_table, user_table)
    return _mm_elu(s, W, b.reshape(1, D))


# speedup vs baseline: 1.1968x; 1.1968x over previous
import jax
import jax.numpy as jnp
from jax import lax
from jax.experimental import pallas as pl
from jax.experimental.pallas import tpu as pltpu
from jax.experimental.pallas import tpu_sc as plsc

B, L, D = 4096, 200, 64
NC, NS = 2, 16
NW = NC * NS
BPW = B // NW
LANES = 16
DV = D // LANES


def _sc_gather_sum_body(item_hbm, user_hbm, idx_hbm, uids_hbm, out_hbm,
                        idx_v, uids_v, self_v, rows0_v, rows1_v, out_v,
                        sem0, sem1, usem):
    wid = lax.axis_index("s") * NC + lax.axis_index("c")
    base = wid * BPW

    pltpu.sync_copy(idx_hbm.at[pl.ds(base, BPW)], idx_v)
    pltpu.sync_copy(uids_hbm.at[pl.ds(base, BPW)], uids_v)
    user_cp = pltpu.async_copy(user_hbm.at[uids_v], self_v, usem)

    def start(b, rows_v, sem):
        pltpu.async_copy(item_hbm.at[idx_v.at[b]], rows_v, sem)

    def accum_row(b, rows_v):
        def accum(j, accs):
            return tuple(accs[d] + rows_v[j, pl.ds(d * LANES, LANES)]
                         for d in range(DV))

        init = tuple(self_v[b, pl.ds(d * LANES, LANES)] for d in range(DV))
        accs = lax.fori_loop(0, L, accum, init, unroll=8)
        for d in range(DV):
            out_v[b, pl.ds(d * LANES, LANES)] = accs[d]

    start(0, rows0_v, sem0)
    user_cp.wait()

    def outer(k, carry):
        start(2 * k + 1, rows1_v, sem1)
        pltpu.make_async_copy(
            item_hbm.at[idx_v.at[0]], rows0_v, sem0).wait()
        accum_row(2 * k, rows0_v)

        @pl.when(k < BPW // 2 - 1)
        def _():
            start(2 * k + 2, rows0_v, sem0)

        pltpu.make_async_copy(
            item_hbm.at[idx_v.at[0]], rows1_v, sem1).wait()
        accum_row(2 * k + 1, rows1_v)
        return carry

    lax.fori_loop(0, BPW // 2, outer, 0)
    pltpu.sync_copy(out_v, out_hbm.at[pl.ds(base, BPW)])


@jax.jit
def _sc_gather_sum(u_item_pad, uids, item_table, user_table):
    mesh = plsc.VectorSubcoreMesh(core_axis_name="c", subcore_axis_name="s")
    return pl.kernel(
        _sc_gather_sum_body,
        out_type=jax.ShapeDtypeStruct((B, D), jnp.float32),
        mesh=mesh,
        scratch_types=[
            pltpu.VMEM((BPW, L), jnp.int32),
            pltpu.VMEM((BPW,), jnp.int32),
            pltpu.VMEM((BPW, D), jnp.float32),
            pltpu.VMEM((L, D), jnp.float32),
            pltpu.VMEM((L, D), jnp.float32),
            pltpu.VMEM((BPW, D), jnp.float32),
            pltpu.SemaphoreType.DMA,
            pltpu.SemaphoreType.DMA,
            pltpu.SemaphoreType.DMA,
        ],
        compiler_params=pltpu.CompilerParams(use_tc_tiling_on_sc=False),
    )(item_table, user_table, u_item_pad, uids)


import functools

RBLK = 2048


def _retile_body(a_ref, b_ref, o_ref):
    o_ref[...] = jnp.concatenate([a_ref[...].T, b_ref[...].T], axis=1)


@functools.partial(jax.jit, static_argnums=(1,))
def _tc_retile(table_t, n_rows):
    nb = (n_rows + RBLK - 1) // RBLK
    max_ib = (n_rows - 1) // (RBLK // 2)
    return pl.pallas_call(
        _retile_body,
        grid=(nb,),
        in_specs=[
            pl.BlockSpec((D, RBLK // 2), lambda i: (0, jnp.minimum(2 * i, max_ib))),
            pl.BlockSpec((D, RBLK // 2), lambda i: (0, jnp.minimum(2 * i + 1, max_ib))),
        ],
        out_specs=pl.BlockSpec((RBLK // 2, 2 * D), lambda i: (i, 0)),
        out_shape=jax.ShapeDtypeStruct((nb * RBLK // 2, 2 * D), jnp.float32),
    )(table_t, table_t)


def _remap_idx(g):
    return ((g >> 11) << 11) | ((g & 1023) << 1) | ((g >> 10) & 1)


def _mm_body(s_ref, w_ref, b_ref, o_ref):
    x = s_ref[...]
    y = lax.dot_general(x, w_ref[...], (((1,), (1,)), ((), ())),
                        preferred_element_type=jnp.float32)
    y = y + b_ref[...]
    o_ref[...] = jnp.where(y > 0, y, jnp.exp(jnp.minimum(y, 0.0)) - 1.0)


@jax.jit
def _mm_elu(s, W, b2d):
    blk = 512
    return pl.pallas_call(
        _mm_body,
        grid=(B // blk,),
        in_specs=[
            pl.BlockSpec((blk, D), lambda i: (i, 0)),
            pl.BlockSpec((D, D), lambda i: (0, 0)),
            pl.BlockSpec((1, D), lambda i: (0, 0)),
        ],
        out_specs=pl.BlockSpec((blk, D), lambda i: (i, 0)),
        out_shape=jax.ShapeDtypeStruct((B, D), jnp.float32),
    )(s, W, b2d)


def kernel(uids, u_item_pad, item_table, user_table, W, b):
    n_items, n_users = item_table.shape[0], user_table.shape[0]
    item_r = _tc_retile(item_table.T, n_items)
    user_r = _tc_retile(user_table.T, n_users)
    item_lin = item_r.reshape(item_r.shape[0] * 2, D)
    user_lin = user_r.reshape(user_r.shape[0] * 2, D)
    idx = _remap_idx(u_item_pad.astype(jnp.int32))
    uid = _remap_idx(uids.astype(jnp.int32))
    s = _sc_gather_sum(idx, uid, item_lin, user_lin)
    return _mm_elu(s, W, b.reshape(1, D))


# speedup vs baseline: 1.7582x; 1.4690x over previous
import jax
import jax.numpy as jnp
from jax import lax
from jax.experimental import pallas as pl
from jax.experimental.pallas import tpu as pltpu
from jax.experimental.pallas import tpu_sc as plsc

B, L, D = 4096, 200, 64
NC, NS = 2, 16
NW = NC * NS
BPW = B // NW
LANES = 16
DV = D // LANES


def _sc_gather_sum_body(item_hbm, user_hbm, idx_hbm, uids_hbm, out_hbm,
                        idx_v, uids_v, self_v, rows0_v, rows1_v, out_v,
                        sem0, sem1, usem):
    wid = lax.axis_index("s") * NC + lax.axis_index("c")
    base = wid * BPW

    pltpu.sync_copy(idx_hbm.at[pl.ds(base, BPW)], idx_v)
    pltpu.sync_copy(uids_hbm.at[pl.ds(base, BPW)], uids_v)
    user_cp = pltpu.async_copy(user_hbm.at[uids_v], self_v, usem)

    def start(b, rows_v, sem):
        pltpu.async_copy(item_hbm.at[idx_v.at[b]], rows_v, sem)

    def accum_row(b, rows_v):
        def accum(j, accs):
            return tuple(accs[d] + rows_v[j, pl.ds(d * LANES, LANES)]
                         for d in range(DV))

        init = tuple(self_v[b, pl.ds(d * LANES, LANES)] for d in range(DV))
        accs = lax.fori_loop(0, L, accum, init, unroll=8)
        for d in range(DV):
            out_v[b, pl.ds(d * LANES, LANES)] = accs[d]

    start(0, rows0_v, sem0)
    user_cp.wait()

    def outer(k, carry):
        start(2 * k + 1, rows1_v, sem1)
        pltpu.make_async_copy(
            item_hbm.at[idx_v.at[0]], rows0_v, sem0).wait()
        accum_row(2 * k, rows0_v)

        @pl.when(k < BPW // 2 - 1)
        def _():
            start(2 * k + 2, rows0_v, sem0)

        pltpu.make_async_copy(
            item_hbm.at[idx_v.at[0]], rows1_v, sem1).wait()
        accum_row(2 * k + 1, rows1_v)
        return carry

    lax.fori_loop(0, BPW // 2, outer, 0)
    pltpu.sync_copy(out_v, out_hbm.at[pl.ds(base, BPW)])


@jax.jit
def _sc_gather_sum(u_item_pad, uids, item_table, user_table):
    mesh = plsc.VectorSubcoreMesh(core_axis_name="c", subcore_axis_name="s")
    return pl.kernel(
        _sc_gather_sum_body,
        out_type=jax.ShapeDtypeStruct((B, D), jnp.float32),
        mesh=mesh,
        scratch_types=[
            pltpu.VMEM((BPW, L), jnp.int32),
            pltpu.VMEM((BPW,), jnp.int32),
            pltpu.VMEM((BPW, D), jnp.float32),
            pltpu.VMEM((L, D), jnp.float32),
            pltpu.VMEM((L, D), jnp.float32),
            pltpu.VMEM((BPW, D), jnp.float32),
            pltpu.SemaphoreType.DMA,
            pltpu.SemaphoreType.DMA,
            pltpu.SemaphoreType.DMA,
        ],
        compiler_params=pltpu.CompilerParams(use_tc_tiling_on_sc=False),
    )(item_table, user_table, u_item_pad, uids)


import functools

RBLK = 8192
HB = RBLK // 2
HBITS = HB.bit_length() - 1


def _retile_body(a_ref, b_ref, o_ref):
    o_ref[...] = jnp.concatenate([a_ref[...].T, b_ref[...].T], axis=1)


@functools.partial(jax.jit, static_argnums=(1,))
def _tc_retile(table_t, n_rows):
    nb = (n_rows + RBLK - 1) // RBLK
    max_ib = (n_rows - 1) // (RBLK // 2)
    return pl.pallas_call(
        _retile_body,
        grid=(nb,),
        in_specs=[
            pl.BlockSpec((D, RBLK // 2), lambda i: (0, jnp.minimum(2 * i, max_ib))),
            pl.BlockSpec((D, RBLK // 2), lambda i: (0, jnp.minimum(2 * i + 1, max_ib))),
        ],
        out_specs=pl.BlockSpec((RBLK // 2, 2 * D), lambda i: (i, 0)),
        out_shape=jax.ShapeDtypeStruct((nb * RBLK // 2, 2 * D), jnp.float32),
    )(table_t, table_t)


def _remap_idx(g):
    return (((g >> (HBITS + 1)) << (HBITS + 1))
            | ((g & (HB - 1)) << 1) | ((g >> HBITS) & 1))


def _mm_body(s_ref, w_ref, b_ref, o_ref):
    x = s_ref[...]
    y = lax.dot_general(x, w_ref[...], (((1,), (1,)), ((), ())),
                        preferred_element_type=jnp.float32)
    y = y + b_ref[...]
    o_ref[...] = jnp.where(y > 0, y, jnp.exp(jnp.minimum(y, 0.0)) - 1.0)


@jax.jit
def _mm_elu(s, W, b2d):
    blk = 512
    return pl.pallas_call(
        _mm_body,
        grid=(B // blk,),
        in_specs=[
            pl.BlockSpec((blk, D), lambda i: (i, 0)),
            pl.BlockSpec((D, D), lambda i: (0, 0)),
            pl.BlockSpec((1, D), lambda i: (0, 0)),
        ],
        out_specs=pl.BlockSpec((blk, D), lambda i: (i, 0)),
        out_shape=jax.ShapeDtypeStruct((B, D), jnp.float32),
    )(s, W, b2d)


def kernel(uids, u_item_pad, item_table, user_table, W, b):
    n_items, n_users = item_table.shape[0], user_table.shape[0]
    item_r = _tc_retile(item_table.T, n_items)
    user_r = _tc_retile(user_table.T, n_users)
    item_lin = item_r.reshape(item_r.shape[0] * 2, D)
    user_lin = user_r.reshape(user_r.shape[0] * 2, D)
    idx = _remap_idx(u_item_pad.astype(jnp.int32))
    uid = _remap_idx(uids.astype(jnp.int32))
    s = _sc_gather_sum(idx, uid, item_lin, user_lin)
    return _mm_elu(s, W, b.reshape(1, D))


# speedup vs baseline: 1.8990x; 1.0801x over previous
import jax
import jax.numpy as jnp
from jax import lax
from jax.experimental import pallas as pl
from jax.experimental.pallas import tpu as pltpu
from jax.experimental.pallas import tpu_sc as plsc

B, L, D = 4096, 200, 64
NC, NS = 2, 16
NW = NC * NS
BPW = B // NW
LANES = 16
DV = D // LANES


def _sc_gather_sum_body(item_hbm, user_hbm, idx_hbm, uids_hbm, out_hbm,
                        idx_v, uids_v, self_v, rows0_v, rows1_v, out_v,
                        sem0, sem1, usem):
    wid = lax.axis_index("s") * NC + lax.axis_index("c")
    base = wid * BPW

    pltpu.sync_copy(idx_hbm.at[pl.ds(base, BPW)], idx_v)
    pltpu.sync_copy(uids_hbm.at[pl.ds(base, BPW)], uids_v)
    user_cp = pltpu.async_copy(user_hbm.at[uids_v], self_v, usem)

    def start(b, rows_v, sem):
        pltpu.async_copy(item_hbm.at[idx_v.at[b]], rows_v, sem)

    def accum_row(b, rows_v):
        def accum(j, accs):
            return tuple(accs[d] + rows_v[j, pl.ds(d * LANES, LANES)]
                         for d in range(DV))

        init = tuple(self_v[b, pl.ds(d * LANES, LANES)] for d in range(DV))
        accs = lax.fori_loop(0, L, accum, init, unroll=8)
        for d in range(DV):
            out_v[b, pl.ds(d * LANES, LANES)] = accs[d]

    start(0, rows0_v, sem0)
    user_cp.wait()

    def outer(k, carry):
        start(2 * k + 1, rows1_v, sem1)
        pltpu.make_async_copy(
            item_hbm.at[idx_v.at[0]], rows0_v, sem0).wait()
        accum_row(2 * k, rows0_v)

        @pl.when(k < BPW // 2 - 1)
        def _():
            start(2 * k + 2, rows0_v, sem0)

        pltpu.make_async_copy(
            item_hbm.at[idx_v.at[0]], rows1_v, sem1).wait()
        accum_row(2 * k + 1, rows1_v)
        return carry

    lax.fori_loop(0, BPW // 2, outer, 0)
    pltpu.sync_copy(out_v, out_hbm.at[pl.ds(base, BPW)])


@jax.jit
def _sc_gather_sum(u_item_pad, uids, item_table, user_table):
    mesh = plsc.VectorSubcoreMesh(core_axis_name="c", subcore_axis_name="s")
    return pl.kernel(
        _sc_gather_sum_body,
        out_type=jax.ShapeDtypeStruct((B, D), jnp.float32),
        mesh=mesh,
        scratch_types=[
            pltpu.VMEM((BPW, L), jnp.int32),
            pltpu.VMEM((BPW,), jnp.int32),
            pltpu.VMEM((BPW, D), jnp.float32),
            pltpu.VMEM((L, D), jnp.float32),
            pltpu.VMEM((L, D), jnp.float32),
            pltpu.VMEM((BPW, D), jnp.float32),
            pltpu.SemaphoreType.DMA,
            pltpu.SemaphoreType.DMA,
            pltpu.SemaphoreType.DMA,
        ],
        compiler_params=pltpu.CompilerParams(use_tc_tiling_on_sc=False),
    )(item_table, user_table, u_item_pad, uids)


import functools

RBLK = 16384
HB = RBLK // 2
HBITS = HB.bit_length() - 1


def _retile_body(a_ref, b_ref, o_ref):
    o_ref[...] = jnp.concatenate([a_ref[...].T, b_ref[...].T], axis=1)


@functools.partial(jax.jit, static_argnums=(1,))
def _tc_retile(table_t, n_rows):
    nb = (n_rows + RBLK - 1) // RBLK
    max_ib = (n_rows - 1) // (RBLK // 2)
    return pl.pallas_call(
        _retile_body,
        grid=(nb,),
        in_specs=[
            pl.BlockSpec((D, RBLK // 2), lambda i: (0, jnp.minimum(2 * i, max_ib))),
            pl.BlockSpec((D, RBLK // 2), lambda i: (0, jnp.minimum(2 * i + 1, max_ib))),
        ],
        out_specs=pl.BlockSpec((RBLK // 2, 2 * D), lambda i: (i, 0)),
        out_shape=jax.ShapeDtypeStruct((nb * RBLK // 2, 2 * D), jnp.float32),
    )(table_t, table_t)


def _remap_idx(g):
    return (((g >> (HBITS + 1)) << (HBITS + 1))
            | ((g & (HB - 1)) << 1) | ((g >> HBITS) & 1))


def _mm_body(s_ref, w_ref, b_ref, o_ref):
    x = s_ref[...]
    y = lax.dot_general(x, w_ref[...], (((1,), (1,)), ((), ())),
                        preferred_element_type=jnp.float32)
    y = y + b_ref[...]
    o_ref[...] = jnp.where(y > 0, y, jnp.exp(jnp.minimum(y, 0.0)) - 1.0)


@jax.jit
def _mm_elu(s, W, b2d):
    blk = 512
    return pl.pallas_call(
        _mm_body,
        grid=(B // blk,),
        in_specs=[
            pl.BlockSpec((blk, D), lambda i: (i, 0)),
            pl.BlockSpec((D, D), lambda i: (0, 0)),
            pl.BlockSpec((1, D), lambda i: (0, 0)),
        ],
        out_specs=pl.BlockSpec((blk, D), lambda i: (i, 0)),
        out_shape=jax.ShapeDtypeStruct((B, D), jnp.float32),
    )(s, W, b2d)


def kernel(uids, u_item_pad, item_table, user_table, W, b):
    n_items, n_users = item_table.shape[0], user_table.shape[0]
    item_r = _tc_retile(item_table.T, n_items)
    user_r = _tc_retile(user_table.T, n_users)
    item_lin = item_r.reshape(item_r.shape[0] * 2, D)
    user_lin = user_r.reshape(user_r.shape[0] * 2, D)
    idx = _remap_idx(u_item_pad.astype(jnp.int32))
    uid = _remap_idx(uids.astype(jnp.int32))
    s = _sc_gather_sum(idx, uid, item_lin, user_lin)
    return _mm_elu(s, W, b.reshape(1, D))


# speedup vs baseline: 1.9501x; 1.0269x over previous
import jax
import jax.numpy as jnp
from jax import lax
from jax.experimental import pallas as pl
from jax.experimental.pallas import tpu as pltpu
from jax.experimental.pallas import tpu_sc as plsc

B, L, D = 4096, 200, 64
NC, NS = 2, 16
NW = NC * NS
BPW = B // NW
LANES = 16
DV = D // LANES


def _sc_gather_sum_body(item_hbm, user_hbm, idx_hbm, uids_hbm, out_hbm,
                        idx_v, uids_v, self_v, rows0_v, rows1_v, out_v,
                        sem0, sem1, usem):
    wid = lax.axis_index("s") * NC + lax.axis_index("c")
    base = wid * BPW

    pltpu.sync_copy(idx_hbm.at[pl.ds(base, BPW)], idx_v)
    pltpu.sync_copy(uids_hbm.at[pl.ds(base, BPW)], uids_v)
    user_cp = pltpu.async_copy(user_hbm.at[uids_v], self_v, usem)

    def start(b, rows_v, sem):
        pltpu.async_copy(item_hbm.at[idx_v.at[b]], rows_v, sem)

    def accum_row(b, rows_v):
        def accum(j, accs):
            return tuple(accs[d] + rows_v[j, pl.ds(d * LANES, LANES)]
                         for d in range(DV))

        init = tuple(self_v[b, pl.ds(d * LANES, LANES)] for d in range(DV))
        accs = lax.fori_loop(0, L, accum, init, unroll=8)
        for d in range(DV):
            out_v[b, pl.ds(d * LANES, LANES)] = accs[d]

    start(0, rows0_v, sem0)
    user_cp.wait()

    def outer(k, carry):
        start(2 * k + 1, rows1_v, sem1)
        pltpu.make_async_copy(
            item_hbm.at[idx_v.at[0]], rows0_v, sem0).wait()
        accum_row(2 * k, rows0_v)

        @pl.when(k < BPW // 2 - 1)
        def _():
            start(2 * k + 2, rows0_v, sem0)

        pltpu.make_async_copy(
            item_hbm.at[idx_v.at[0]], rows1_v, sem1).wait()
        accum_row(2 * k + 1, rows1_v)
        return carry

    lax.fori_loop(0, BPW // 2, outer, 0)
    pltpu.sync_copy(out_v, out_hbm.at[pl.ds(base, BPW)])


@jax.jit
def _sc_gather_sum(u_item_pad, uids, item_table, user_table):
    mesh = plsc.VectorSubcoreMesh(core_axis_name="c", subcore_axis_name="s")
    return pl.kernel(
        _sc_gather_sum_body,
        out_type=jax.ShapeDtypeStruct((B, D), jnp.float32),
        mesh=mesh,
        scratch_types=[
            pltpu.VMEM((BPW, L), jnp.int32),
            pltpu.VMEM((BPW,), jnp.int32),
            pltpu.VMEM((BPW, D), jnp.float32),
            pltpu.VMEM((L, D), jnp.float32),
            pltpu.VMEM((L, D), jnp.float32),
            pltpu.VMEM((BPW, D), jnp.float32),
            pltpu.SemaphoreType.DMA,
            pltpu.SemaphoreType.DMA,
            pltpu.SemaphoreType.DMA,
        ],
        compiler_params=pltpu.CompilerParams(use_tc_tiling_on_sc=False),
    )(item_table, user_table, u_item_pad, uids)


import functools

RBLK = 32768
HB = RBLK // 2
HBITS = HB.bit_length() - 1


def _retile_body(a_ref, b_ref, o_ref):
    o_ref[...] = jnp.concatenate([a_ref[...].T, b_ref[...].T], axis=1)


@functools.partial(jax.jit, static_argnums=(1,))
def _tc_retile(table_t, n_rows):
    nb = (n_rows + RBLK - 1) // RBLK
    max_ib = (n_rows - 1) // (RBLK // 2)
    return pl.pallas_call(
        _retile_body,
        grid=(nb,),
        in_specs=[
            pl.BlockSpec((D, RBLK // 2), lambda i: (0, jnp.minimum(2 * i, max_ib))),
            pl.BlockSpec((D, RBLK // 2), lambda i: (0, jnp.minimum(2 * i + 1, max_ib))),
        ],
        out_specs=pl.BlockSpec((RBLK // 2, 2 * D), lambda i: (i, 0)),
        out_shape=jax.ShapeDtypeStruct((nb * RBLK // 2, 2 * D), jnp.float32),
    )(table_t, table_t)


def _remap_idx(g):
    return (((g >> (HBITS + 1)) << (HBITS + 1))
            | ((g & (HB - 1)) << 1) | ((g >> HBITS) & 1))


def _mm_body(s_ref, w_ref, b_ref, o_ref):
    x = s_ref[...]
    y = lax.dot_general(x, w_ref[...], (((1,), (1,)), ((), ())),
                        preferred_element_type=jnp.float32)
    y = y + b_ref[...]
    o_ref[...] = jnp.where(y > 0, y, jnp.exp(jnp.minimum(y, 0.0)) - 1.0)


@jax.jit
def _mm_elu(s, W, b2d):
    blk = 512
    return pl.pallas_call(
        _mm_body,
        grid=(B // blk,),
        in_specs=[
            pl.BlockSpec((blk, D), lambda i: (i, 0)),
            pl.BlockSpec((D, D), lambda i: (0, 0)),
            pl.BlockSpec((1, D), lambda i: (0, 0)),
        ],
        out_specs=pl.BlockSpec((blk, D), lambda i: (i, 0)),
        out_shape=jax.ShapeDtypeStruct((B, D), jnp.float32),
    )(s, W, b2d)


def kernel(uids, u_item_pad, item_table, user_table, W, b):
    n_items, n_users = item_table.shape[0], user_table.shape[0]
    item_r = _tc_retile(item_table.T, n_items)
    user_r = _tc_retile(user_table.T, n_users)
    item_lin = item_r.reshape(item_r.shape[0] * 2, D)
    user_lin = user_r.reshape(user_r.shape[0] * 2, D)
    idx = _remap_idx(u_item_pad.astype(jnp.int32))
    uid = _remap_idx(uids.astype(jnp.int32))
    s = _sc_gather_sum(idx, uid, item_lin, user_lin)
    return _mm_elu(s, W, b.reshape(1, D))
